# Initial kernel scaffold; baseline (speedup 1.0000x reference)
import jax
import jax.numpy as jnp
from jax.experimental import pallas as pl


def kernel(batch, emb_weight):
    raise NotImplementedError("write your pallas kernel here")


# speedup vs baseline: 3.5314x; 3.5314x over previous
import functools

import jax
import jax.numpy as jnp
from jax import lax
from jax.experimental import pallas as pl
from jax.experimental.pallas import tpu as pltpu
from jax.experimental.pallas import tpu_sc as plsc


@functools.lru_cache(maxsize=None)
def _make_gather(V, D, B):
    info = plsc.get_sparse_core_info()
    NC, NS, L = info.num_cores, info.num_subcores, info.num_lanes
    NW = NC * NS
    assert B % NW == 0
    b_per_w = B // NW
    CHUNK = 128
    assert b_per_w % CHUNK == 0
    n_chunks = b_per_w // CHUNK
    DM = D - 1

    mesh = plsc.VectorSubcoreMesh(core_axis_name="c", subcore_axis_name="s")

    @functools.partial(
        pl.kernel,
        mesh=mesh,
        out_type=(
            jax.ShapeDtypeStruct((B, DM), jnp.float32),
            jax.ShapeDtypeStruct((B,), jnp.float32),
        ),
        scratch_types=[
            pltpu.VMEM((b_per_w,), jnp.int32),
            pltpu.VMEM((b_per_w, DM), jnp.float32),
            pltpu.VMEM((b_per_w,), jnp.float32),
            pltpu.VMEM((CHUNK, DM), jnp.float32),
            pltpu.VMEM((CHUNK, DM), jnp.float32),
            pltpu.SemaphoreType.DMA,
            pltpu.SemaphoreType.DMA,
        ],
    )
    def k(table_hbm, idx_hbm, out_main_hbm, out_last_hbm,
          idx_v, main_v, last_v, aux0_v, aux1_v, sem_m, sem_a):
        wid = lax.axis_index("s") * NC + lax.axis_index("c")
        base = wid * b_per_w
        pltpu.sync_copy(idx_hbm.at[pl.ds(base, b_per_w)], idx_v)

        main_view = table_hbm.at[:, pl.ds(0, DM)]
        aux_start = pl.multiple_of(wid * 0 + DM, DM)
        aux_view = table_hbm.at[:, pl.ds(aux_start, DM)]

        main_copies = [
            pltpu.async_copy(
                main_view.at[idx_v.at[pl.ds(j * CHUNK, CHUNK)]],
                main_v.at[pl.ds(j * CHUNK, CHUNK)], sem_m)
            for j in range(n_chunks)
        ]
        aux_bufs = [aux0_v, aux1_v]
        aux_copies = [None] * n_chunks
        for j in range(min(2, n_chunks)):
            aux_copies[j] = pltpu.async_copy(
                aux_view.at[idx_v.at[pl.ds(j * CHUNK, CHUNK)]],
                aux_bufs[j % 2], sem_a)
        for j in range(n_chunks):
            aux_copies[j].wait()
            buf = aux_bufs[j % 2]
            lane = lax.iota(jnp.int32, L)
            for g in range(CHUNK // L):
                acc = jnp.zeros((L,), jnp.float32)
                for t in range(L):
                    v = buf[g * L + t, pl.ds(0, L)]
                    acc = jnp.where(lane == t, v[0], acc)
                last_v[pl.ds(j * CHUNK + g * L, L)] = acc
            nxt = j + 2
            if nxt < n_chunks:
                aux_copies[nxt] = pltpu.async_copy(
                    aux_view.at[idx_v.at[pl.ds(nxt * CHUNK, CHUNK)]],
                    aux_bufs[nxt % 2], sem_a)

        for c in main_copies:
            c.wait()
        pltpu.sync_copy(main_v, out_main_hbm.at[pl.ds(base, b_per_w)])
        pltpu.sync_copy(last_v, out_last_hbm.at[pl.ds(base, b_per_w)])

    return k


def kernel(batch, emb_weight):
    V, D = emb_weight.shape
    (B,) = batch.shape
    main, last = _make_gather(V, D, B)(emb_weight, batch.astype(jnp.int32))
    return jnp.concatenate([main, last[:, None]], axis=1)


# speedup vs baseline: 4.0411x; 1.1443x over previous
import functools

import jax
import jax.numpy as jnp
from jax import lax
from jax.experimental import pallas as pl
from jax.experimental.pallas import tpu as pltpu
from jax.experimental.pallas import tpu_sc as plsc


@functools.lru_cache(maxsize=None)
def _make_gather(V, D, B):
    info = plsc.get_sparse_core_info()
    NC, NS, L = info.num_cores, info.num_subcores, info.num_lanes
    NW = NC * NS
    assert B % NW == 0
    b_per_w = B // NW
    CHUNK = 128
    assert b_per_w % CHUNK == 0
    n_chunks = b_per_w // CHUNK
    DM = D - 1

    mesh = plsc.VectorSubcoreMesh(core_axis_name="c", subcore_axis_name="s")

    @functools.partial(
        pl.kernel,
        mesh=mesh,
        out_type=jax.ShapeDtypeStruct((B, D), jnp.float32),
        scratch_types=[
            pltpu.VMEM((b_per_w,), jnp.int32),
            pltpu.VMEM((CHUNK, D), jnp.float32),
            pltpu.VMEM((CHUNK, D), jnp.float32),
            pltpu.VMEM((CHUNK, DM), jnp.float32),
            pltpu.VMEM((CHUNK, DM), jnp.float32),
            pltpu.SemaphoreType.DMA,
            pltpu.SemaphoreType.DMA,
            pltpu.SemaphoreType.DMA,
        ],
    )
    def k(table_hbm, idx_hbm, out_hbm,
          idx_v, row0_v, row1_v, aux0_v, aux1_v, sem_m, sem_a, sem_w):
        wid = lax.axis_index("s") * NC + lax.axis_index("c")
        base = wid * b_per_w
        pltpu.sync_copy(idx_hbm.at[pl.ds(base, b_per_w)], idx_v)

        main_view = table_hbm.at[:, pl.ds(0, DM)]
        aux_start = pl.multiple_of(wid * 0 + DM, DM)
        aux_view = table_hbm.at[:, pl.ds(aux_start, DM)]
        row_bufs = [row0_v, row1_v]
        aux_bufs = [aux0_v, aux1_v]
        lane = lax.iota(jnp.int32, L)

        main_copies = [None] * n_chunks
        aux_copies = [None] * n_chunks
        write_copies = [None] * n_chunks

        def fire(j):
            isl = idx_v.at[pl.ds(j * CHUNK, CHUNK)]
            main_copies[j] = pltpu.async_copy(
                main_view.at[isl], row_bufs[j % 2].at[:, pl.ds(0, DM)],
                sem_m)
            aux_copies[j] = pltpu.async_copy(
                aux_view.at[isl], aux_bufs[j % 2], sem_a)

        for j in range(min(2, n_chunks)):
            fire(j)
        for j in range(n_chunks):
            main_copies[j].wait()
            aux_copies[j].wait()
            rbuf = row_bufs[j % 2]
            abuf = aux_bufs[j % 2]
            col = pl.multiple_of(wid * 0 + DM, DM)
            for r in range(CHUNK):
                v = abuf[r, pl.ds(0, L)]
                rbuf[r, pl.ds(col, L)] = jnp.where(lane >= 0, v[0], 0.0)
            write_copies[j] = pltpu.async_copy(
                rbuf, out_hbm.at[pl.ds(base + j * CHUNK, CHUNK)], sem_w)
            nxt = j + 2
            if nxt < n_chunks:
                write_copies[j].wait()
                fire(nxt)
        for j in range(n_chunks):
            if j >= n_chunks - 2:
                write_copies[j].wait()

    return k


def kernel(batch, emb_weight):
    V, D = emb_weight.shape
    (B,) = batch.shape
    return _make_gather(V, D, B)(emb_weight, batch.astype(jnp.int32))


# speedup vs baseline: 4.1018x; 1.0150x over previous
import functools

import jax
import jax.numpy as jnp
from jax import lax
from jax.experimental import pallas as pl
from jax.experimental.pallas import tpu as pltpu
from jax.experimental.pallas import tpu_sc as plsc


@functools.lru_cache(maxsize=None)
def _make_gather(V, D, B):
    info = plsc.get_sparse_core_info()
    NC, NS, L = info.num_cores, info.num_subcores, info.num_lanes
    NW = NC * NS
    assert B % NW == 0
    b_per_w = B // NW
    CHUNK = 128
    assert b_per_w % CHUNK == 0
    n_chunks = b_per_w // CHUNK
    DP = 2 * 128
    NBUF = 3

    mesh = plsc.VectorSubcoreMesh(core_axis_name="c", subcore_axis_name="s")

    @functools.partial(
        pl.kernel,
        mesh=mesh,
        out_type=jax.ShapeDtypeStruct((B, D), jnp.float32),
        scratch_types=[
            pltpu.VMEM((b_per_w,), jnp.int32),
            pltpu.VMEM((CHUNK, D), jnp.float32),
            pltpu.VMEM((CHUNK, D), jnp.float32),
            pltpu.VMEM((CHUNK, D), jnp.float32),
            pltpu.SemaphoreType.DMA,
            pltpu.SemaphoreType.DMA,
        ],
    )
    def k(table_hbm, idx_hbm, out_hbm,
          idx_v, buf0_v, buf1_v, buf2_v, sem_g, sem_w):
        wid = lax.axis_index("s") * NC + lax.axis_index("c")
        base = wid * b_per_w
        pltpu.sync_copy(idx_hbm.at[pl.ds(base, b_per_w)], idx_v)

        start = pl.multiple_of(wid * 0, DP)
        row_view = table_hbm.at[:, pl.ds(start, DP)]
        bufs = [buf0_v, buf1_v, buf2_v]

        gathers = [None] * n_chunks
        writes = [None] * n_chunks

        def fire(j):
            gathers[j] = pltpu.async_copy(
                row_view.at[idx_v.at[pl.ds(j * CHUNK, CHUNK)]],
                bufs[j % NBUF].at[:, pl.ds(start, DP)], sem_g)

        for j in range(min(NBUF, n_chunks)):
            fire(j)
        for j in range(n_chunks):
            gathers[j].wait()
            writes[j] = pltpu.async_copy(
                bufs[j % NBUF],
                out_hbm.at[pl.ds(base + j * CHUNK, CHUNK)], sem_w)
            nxt = j + NBUF
            if nxt < n_chunks:
                writes[j].wait()
                fire(nxt)
        for j in range(max(0, n_chunks - NBUF), n_chunks):
            writes[j].wait()

    return k


def kernel(batch, emb_weight):
    V, D = emb_weight.shape
    (B,) = batch.shape
    return _make_gather(V, D, B)(emb_weight, batch.astype(jnp.int32))
